# baseline (speedup 1.0000x reference)
import functools

import jax
import jax.numpy as jnp
from jax import lax
from jax.experimental import pallas as pl
from jax.experimental.pallas import tpu as pltpu
from jax.experimental.pallas import tpu_sc as plsc

_N = 10000
_E = 320000
_C = 128

_NC = 2
_NS = 16
_NW = _NC * _NS
_CH = 128
_EPW = _E // _NW
_EFULL = _EPW // _CH
_ETAIL = _EPW - _EFULL * _CH
_NFULL = _N // _CH
_NTAIL = _N - _NFULL * _CH

_BE = 2000

_mesh = plsc.VectorSubcoreMesh(core_axis_name="c", subcore_axis_name="s")


@functools.partial(
    pl.kernel, mesh=_mesh,
    out_type=[jax.ShapeDtypeStruct((_E, _C), jnp.float32),
              jax.ShapeDtypeStruct((_E, _C), jnp.float32)],
    scratch_types=[pltpu.VMEM((_CH,), jnp.int32),
                   pltpu.VMEM((_CH,), jnp.int32),
                   pltpu.VMEM((_CH,), jnp.int32),
                   pltpu.VMEM((_CH,), jnp.int32),
                   pltpu.VMEM((_ETAIL,), jnp.int32),
                   pltpu.VMEM((_ETAIL,), jnp.int32),
                   pltpu.VMEM((_CH, _C), jnp.float32),
                   pltpu.VMEM((_CH, _C), jnp.float32),
                   pltpu.VMEM((_CH, _C), jnp.float32),
                   pltpu.VMEM((_CH, _C), jnp.float32),
                   pltpu.VMEM((_ETAIL, _C), jnp.float32),
                   pltpu.VMEM((_ETAIL, _C), jnp.float32),
                   pltpu.SemaphoreType.DMA,
                   pltpu.SemaphoreType.DMA],
)
def _sc_gather2(x_hbm, src_hbm, dst_hbm, outs_hbm, outd_hbm,
                si0, di0, si1, di1, si_t, di_t,
                sr0, dr0, sr1, dr1, sr_t, dr_t, sem0, sem1):
    wid = lax.axis_index("s") * _NC + lax.axis_index("c")
    base_w = pl.multiple_of(wid * _EPW, 8)
    sis = (si0, si1)
    dis = (di0, di1)
    srs = (sr0, sr1)
    drs = (dr0, dr1)
    sems = (sem0, sem1)

    def _start(c, slot):
        base = pl.multiple_of(base_w + c * _CH, 8)
        pltpu.sync_copy(src_hbm.at[pl.ds(base, _CH)], sis[slot])
        pltpu.sync_copy(dst_hbm.at[pl.ds(base, _CH)], dis[slot])
        pltpu.async_copy(x_hbm.at[sis[slot]], srs[slot], sems[slot])
        pltpu.async_copy(x_hbm.at[dis[slot]], drs[slot], sems[slot])

    def _finish(c, slot):
        pltpu.make_async_copy(x_hbm.at[sis[slot]], srs[slot], sems[slot]).wait()
        pltpu.make_async_copy(x_hbm.at[dis[slot]], drs[slot], sems[slot]).wait()
        base = pl.multiple_of(base_w + c * _CH, 8)
        pltpu.sync_copy(srs[slot], outs_hbm.at[pl.ds(base, _CH)])
        pltpu.sync_copy(drs[slot], outd_hbm.at[pl.ds(base, _CH)])

    for slot in (0, 1):
        _start(slot, slot)

    def body(t, carry):
        for slot in (0, 1):
            c = 2 * t + slot
            _finish(c, slot)
            _start(c + 2, slot)
        return carry

    lax.fori_loop(0, (_EFULL - 2) // 2, body, 0)
    for slot in (0, 1):
        _finish(_EFULL - 2 + slot, slot)

    base = pl.multiple_of(base_w + _EFULL * _CH, 8)
    pltpu.sync_copy(src_hbm.at[pl.ds(base, _ETAIL)], si_t)
    pltpu.sync_copy(dst_hbm.at[pl.ds(base, _ETAIL)], di_t)
    cs = pltpu.async_copy(x_hbm.at[si_t], sr_t, sem0)
    cd = pltpu.async_copy(x_hbm.at[di_t], dr_t, sem0)
    cs.wait()
    cd.wait()
    pltpu.sync_copy(sr_t, outs_hbm.at[pl.ds(base, _ETAIL)])
    pltpu.sync_copy(dr_t, outd_hbm.at[pl.ds(base, _ETAIL)])


def _make_sc_scatter(with_values):
    scratch = [pltpu.VMEM((_CH,), jnp.int32),
               pltpu.VMEM((_CH,), jnp.int32),
               pltpu.VMEM((_ETAIL,), jnp.int32),
               pltpu.VMEM((_CH, _C), jnp.float32),
               pltpu.VMEM((_CH, _C), jnp.float32),
               pltpu.VMEM((_ETAIL, _C), jnp.float32),
               pltpu.VMEM_SHARED((_N, _C), jnp.float32),
               pltpu.SemaphoreType.DMA,
               pltpu.SemaphoreType.DMA]

    @functools.partial(
        pl.kernel, mesh=_mesh,
        out_type=[jax.ShapeDtypeStruct((_NC * _N, _C), jnp.float32)],
        scratch_types=scratch)
    def k(h_hbm, dst_hbm, z128_hbm, o128_hbm, agg_hbm,
          di0, di1, di_t, hv0, hv1, hv_t, acc_sh, sem0, sem1):
        cid = lax.axis_index("c")
        sid = lax.axis_index("s")
        wid = sid * _NC + cid
        dis = (di0, di1)
        hvs = (hv0, hv1)
        sems = (sem0, sem1)

        pltpu.sync_copy(z128_hbm, hv0)
        _RPT = 624
        _NT_FULL = _RPT // _CH
        _NT_TAIL = _RPT - _NT_FULL * _CH
        _GTB = _NS * _RPT
        nbase = sid * _RPT

        def zbody(t, carry):
            b = nbase + t * _CH
            pltpu.sync_copy(hv0, acc_sh.at[pl.ds(b, _CH)])
            return carry

        lax.fori_loop(0, _NT_FULL, zbody, 0)
        tb = nbase + _NT_FULL * _CH
        pltpu.sync_copy(hv0.at[pl.ds(0, _NT_TAIL)], acc_sh.at[pl.ds(tb, _NT_TAIL)])
        pltpu.sync_copy(hv0.at[pl.ds(0, _NTAIL)], acc_sh.at[pl.ds(_GTB, _NTAIL)])
        if not with_values:
            pltpu.sync_copy(o128_hbm, hv0)
            pltpu.sync_copy(o128_hbm, hv1)
            pltpu.sync_copy(o128_hbm.at[pl.ds(0, _ETAIL)], hv_t)

        plsc.subcore_barrier()

        base_w = pl.multiple_of(wid * _EPW, 8)

        def _stage(c, slot):
            base = pl.multiple_of(base_w + c * _CH, 8)
            pltpu.sync_copy(dst_hbm.at[pl.ds(base, _CH)], dis[slot])
            if with_values:
                pltpu.async_copy(h_hbm.at[pl.ds(base, _CH)], hvs[slot],
                                 sems[slot])

        def _commit(slot):
            if with_values:
                pltpu.make_async_copy(
                    h_hbm.at[pl.ds(0, _CH)], hvs[slot], sems[slot]).wait()
            pltpu.sync_copy(hvs[slot], acc_sh.at[dis[slot]], add=True)

        for slot in (0, 1):
            _stage(slot, slot)

        def body(t, carry):
            for slot in (0, 1):
                _commit(slot)
                _stage(2 * t + slot + 2, slot)
            return carry

        lax.fori_loop(0, (_EFULL - 2) // 2, body, 0)
        for slot in (0, 1):
            _commit(slot)

        base = pl.multiple_of(base_w + _EFULL * _CH, 8)
        pltpu.sync_copy(dst_hbm.at[pl.ds(base, _ETAIL)], di_t)
        if with_values:
            pltpu.sync_copy(h_hbm.at[pl.ds(base, _ETAIL)], hv_t)
        pltpu.sync_copy(hv_t, acc_sh.at[di_t], add=True)

        plsc.subcore_barrier()

        def obody(t, carry):
            b = nbase + t * _CH
            ob = cid * _N + b
            pltpu.sync_copy(acc_sh.at[pl.ds(b, _CH)], hv0)
            pltpu.sync_copy(hv0, agg_hbm.at[pl.ds(ob, _CH)])
            return carry

        lax.fori_loop(0, _NT_FULL, obody, 0)
        ob = cid * _N + tb
        pltpu.sync_copy(acc_sh.at[pl.ds(tb, _NT_TAIL)], hv0.at[pl.ds(0, _NT_TAIL)])
        pltpu.sync_copy(hv0.at[pl.ds(0, _NT_TAIL)], agg_hbm.at[pl.ds(ob, _NT_TAIL)])
        gob = cid * _N + _GTB
        pltpu.sync_copy(acc_sh.at[pl.ds(_GTB, _NTAIL)], hv_t)
        pltpu.sync_copy(hv_t, agg_hbm.at[pl.ds(gob, _NTAIL)])

    return k


def _unwrap(res):
    return res[0] if isinstance(res, (list, tuple)) else res


_sc_scatter_vals_raw = _make_sc_scatter(True)
_sc_deg_raw = _make_sc_scatter(False)


def _sc_scatter_vals(*a):
    return _unwrap(_sc_scatter_vals_raw(*a))


def _sc_deg(*a):
    return _unwrap(_sc_deg_raw(*a))


def _full_spec(arr):
    nd = arr.ndim
    return pl.BlockSpec(arr.shape, lambda i, _n=nd: (0,) * _n)


def _edge_spec(width):
    return pl.BlockSpec((_BE, width), lambda i: (i, 0))


def _bdot(a, w):
    return jnp.dot(a.astype(jnp.bfloat16), w,
                   preferred_element_type=jnp.float32)


def _mlp_tail(h, wrefs):
    n = len(wrefs) // 2
    for i in range(n):
        w = wrefs[2 * i][...]
        b = wrefs[2 * i + 1][...]
        h = _bdot(h, w) + b
        if i < n - 1:
            h = jnp.maximum(h, 0.0)
    return h


def _split_first(params, widths):
    w0 = params[0]
    parts = []
    off = 0
    for w in widths:
        parts.append(w0[off:off + w])
        off += w
    parts = [p.astype(jnp.bfloat16) for p in parts]
    rest = []
    for i in range(1, len(params)):
        p = params[i]
        rest.append(p.reshape(1, -1) if p.ndim == 1 else p.astype(jnp.bfloat16))
    return parts, rest


def _nc_mlp(xs, xd, ang, params):
    (ws, wd, wa), rest = _split_first(params, (_C, _C, 1))
    b0, tail = rest[0], rest[1:]
    ins = [xs, xd, ang, ws, wd, wa, b0] + tail
    n_tail = len(tail)

    def body(*refs):
        xs_r, xd_r, an_r, ws_r, wd_r, wa_r, b0_r = refs[:7]
        wrefs = refs[7:7 + n_tail]
        out_r = refs[7 + n_tail]
        h = (_bdot(xs_r[...], ws_r[...]) + _bdot(xd_r[...], wd_r[...])
             + an_r[...] * wa_r[...].astype(jnp.float32) + b0_r[...])
        h = jnp.maximum(h, 0.0)
        out_r[...] = _mlp_tail(h, wrefs)

    return pl.pallas_call(
        body,
        grid=(_E // _BE,),
        in_specs=[_edge_spec(_C), _edge_spec(_C), _edge_spec(1)]
                 + [_full_spec(a) for a in ins[3:]],
        out_specs=_edge_spec(_C),
        out_shape=jax.ShapeDtypeStruct((_E, _C), jnp.float32),
        compiler_params=pltpu.CompilerParams(
            dimension_semantics=("arbitrary",)),
    )(*ins)


def _fused_ec1_nc2(xs, xd, ang, act, ec1_params, nc2_params):
    (ews, ewd, ewa), erest = _split_first(ec1_params, (_C, _C, 1))
    eb0, etail = erest[0], erest[1:]
    (nws, nwd, nwa), nrest = _split_first(nc2_params, (_C, _C, 1))
    nb0, ntail = nrest[0], nrest[1:]
    ins = ([xs, xd, ang, act, ews, ewd, ewa, eb0] + etail
           + [nws, nwd, nwa, nb0] + ntail)
    ne, nn = len(etail), len(ntail)

    def body(*refs):
        xs_r, xd_r, an_r, ac_r = refs[:4]
        ews_r, ewd_r, ewa_r, eb0_r = refs[4:8]
        ewrefs = refs[8:8 + ne]
        nws_r, nwd_r, nwa_r, nb0_r = refs[8 + ne:12 + ne]
        nwrefs = refs[12 + ne:12 + ne + nn]
        e1_r, h2_r, ss_r = refs[12 + ne + nn:]
        xs_v = xs_r[...]
        xd_v = xd_r[...]
        xs_b = xs_v.astype(jnp.bfloat16)
        xd_b = xd_v.astype(jnp.bfloat16)
        he = (_bdot(xs_b, ews_r[...]) + _bdot(xd_b, ewd_r[...])
              + ac_r[...] * ewa_r[...].astype(jnp.float32) + eb0_r[...])
        he = jnp.maximum(he, 0.0)
        e1 = _mlp_tail(he, ewrefs)
        e1_r[...] = e1.astype(jnp.bfloat16)
        hn = (_bdot(xs_b, nws_r[...]) + _bdot(xd_b, nwd_r[...])
              + an_r[...] * nwa_r[...].astype(jnp.float32) + nb0_r[...])
        hn = jnp.maximum(hn, 0.0)
        h2_r[...] = _mlp_tail(hn, nwrefs)

        @pl.when(pl.program_id(0) == 0)
        def _():
            ss_r[...] = jnp.zeros((8, 128), jnp.float32)

        ss_r[...] += jnp.sum(e1 * e1)

    h_ec = ec1_params[-1].shape[0]
    return pl.pallas_call(
        body,
        grid=(_E // _BE,),
        in_specs=[_edge_spec(_C), _edge_spec(_C), _edge_spec(1), _edge_spec(1)]
                 + [_full_spec(a) for a in ins[4:]],
        out_specs=[_edge_spec(h_ec), _edge_spec(_C),
                   pl.BlockSpec((8, 128), lambda i: (0, 0))],
        out_shape=[jax.ShapeDtypeStruct((_E, h_ec), jnp.bfloat16),
                   jax.ShapeDtypeStruct((_E, _C), jnp.float32),
                   jax.ShapeDtypeStruct((8, 128), jnp.float32)],
        compiler_params=pltpu.CompilerParams(
            dimension_semantics=("arbitrary",)),
    )(*ins)


def _ec2_mlp(xs, xd, e1, params):
    h_in = e1.shape[1]
    (ws, wd, we), rest = _split_first(params, (_C, _C, h_in))
    b0, tail = rest[0], rest[1:]
    ins = [xs, xd, e1, ws, wd, we, b0] + tail
    n_tail = len(tail)

    def body(*refs):
        xs_r, xd_r, e1_r, ws_r, wd_r, we_r, b0_r = refs[:7]
        wrefs = refs[7:7 + n_tail]
        out_r, ss_r = refs[7 + n_tail:]
        h = (_bdot(xs_r[...], ws_r[...]) + _bdot(xd_r[...], wd_r[...])
             + _bdot(e1_r[...], we_r[...]) + b0_r[...])
        h = jnp.maximum(h, 0.0)
        e2 = _mlp_tail(h, wrefs)
        out_r[...] = e2

        @pl.when(pl.program_id(0) == 0)
        def _():
            ss_r[...] = jnp.zeros((8, 128), jnp.float32)

        ss_r[...] += jnp.sum(e2 * e2)

    return pl.pallas_call(
        body,
        grid=(_E // _BE,),
        in_specs=[_edge_spec(_C), _edge_spec(_C), _edge_spec(h_in)]
                 + [_full_spec(a) for a in ins[3:]],
        out_specs=[_edge_spec(_C), pl.BlockSpec((8, 128), lambda i: (0, 0))],
        out_shape=[jax.ShapeDtypeStruct((_E, _C), jnp.float32),
                   jax.ShapeDtypeStruct((8, 128), jnp.float32)],
        compiler_params=pltpu.CompilerParams(
            dimension_semantics=("arbitrary",)),
    )(*ins)


_BN = 1000


def _node_update(x, agg_parts, deg_parts):
    nb = _N // _BN

    def body(x_r, a0_r, a1_r, d0_r, d1_r, out_r):
        deg = d0_r[...][:, 0:1] + d1_r[...][:, 0:1]
        deg = jnp.maximum(deg, 1.0)
        agg = a0_r[...] + a1_r[...]
        out_r[...] = x_r[...] + jnp.maximum(agg / deg, 0.0)

    return pl.pallas_call(
        body,
        grid=(nb,),
        in_specs=[pl.BlockSpec((_BN, _C), lambda i: (i, 0)),
                  pl.BlockSpec((_BN, _C), lambda i: (i, 0)),
                  pl.BlockSpec((_BN, _C), lambda i: (i + nb, 0)),
                  pl.BlockSpec((_BN, _C), lambda i: (i, 0)),
                  pl.BlockSpec((_BN, _C), lambda i: (i + nb, 0))],
        out_specs=pl.BlockSpec((_BN, _C), lambda i: (i, 0)),
        out_shape=jax.ShapeDtypeStruct((_N, _C), jnp.float32),
        compiler_params=pltpu.CompilerParams(
            dimension_semantics=("arbitrary",)),
    )(x, agg_parts, agg_parts, deg_parts, deg_parts)


def kernel(node_features, edge_index, angles, gt_edges, actions,
           nc1_params, ec1_params, nc2_params, ec2_params):
    src = edge_index[0]
    dst = edge_index[1]
    ang = angles.reshape(_E, 1)
    z128 = jnp.zeros((_CH, _C), jnp.float32)
    o128 = jnp.ones((_CH, _C), jnp.float32)

    x0 = node_features
    deg = _sc_deg(z128, dst, z128, o128)

    xs0, xd0 = _sc_gather2(x0, src, dst)
    h1 = _nc_mlp(xs0, xd0, ang, nc1_params)
    agg1 = _sc_scatter_vals(h1, dst, z128, o128)
    x1 = _node_update(x0, agg1, deg)

    xs1, xd1 = _sc_gather2(x1, src, dst)
    e1, h2, ss1 = _fused_ec1_nc2(xs1, xd1, ang, actions, ec1_params,
                                 nc2_params)
    agg2 = _sc_scatter_vals(h2, dst, z128, o128)
    x2 = _node_update(x1, agg2, deg)

    xs2, xd2 = _sc_gather2(x2, src, dst)
    e2, ss2 = _ec2_mlp(xs2, xd2, e1, ec2_params)

    h_ec = e1.shape[1]
    side = (ss1[0, 0] / (_E * float(h_ec)) + ss2[0, 0] / (_E * float(_C))) * 0.5
    return e2, side


# speedup vs baseline: 1.1071x; 1.1071x over previous
import functools

import jax
import jax.numpy as jnp
from jax import lax
from jax.experimental import pallas as pl
from jax.experimental.pallas import tpu as pltpu
from jax.experimental.pallas import tpu_sc as plsc

_N = 10000
_E = 320000
_C = 128

_NC = 2
_NS = 16
_NW = _NC * _NS
_CH = 128
_EPW = _E // _NW
_EFULL = _EPW // _CH
_ETAIL = _EPW - _EFULL * _CH
_NFULL = _N // _CH
_NTAIL = _N - _NFULL * _CH

_BE = 2000

_mesh = plsc.VectorSubcoreMesh(core_axis_name="c", subcore_axis_name="s")


@functools.partial(
    pl.kernel, mesh=_mesh,
    out_type=[jax.ShapeDtypeStruct((_E, _C), jnp.float32),
              jax.ShapeDtypeStruct((_E, _C), jnp.float32)],
    scratch_types=[pltpu.VMEM((_CH,), jnp.int32),
                   pltpu.VMEM((_CH,), jnp.int32),
                   pltpu.VMEM((_CH,), jnp.int32),
                   pltpu.VMEM((_CH,), jnp.int32),
                   pltpu.VMEM((_ETAIL,), jnp.int32),
                   pltpu.VMEM((_ETAIL,), jnp.int32),
                   pltpu.VMEM((_CH, _C), jnp.float32),
                   pltpu.VMEM((_CH, _C), jnp.float32),
                   pltpu.VMEM((_CH, _C), jnp.float32),
                   pltpu.VMEM((_CH, _C), jnp.float32),
                   pltpu.VMEM((_ETAIL, _C), jnp.float32),
                   pltpu.VMEM((_ETAIL, _C), jnp.float32),
                   pltpu.SemaphoreType.DMA,
                   pltpu.SemaphoreType.DMA],
)
def _sc_gather2(x_hbm, src_hbm, dst_hbm, outs_hbm, outd_hbm,
                si0, di0, si1, di1, si_t, di_t,
                sr0, dr0, sr1, dr1, sr_t, dr_t, sem0, sem1):
    wid = lax.axis_index("s") * _NC + lax.axis_index("c")
    base_w = pl.multiple_of(wid * _EPW, 8)
    sis = (si0, si1)
    dis = (di0, di1)
    srs = (sr0, sr1)
    drs = (dr0, dr1)
    sems = (sem0, sem1)

    def _start(c, slot):
        base = pl.multiple_of(base_w + c * _CH, 8)
        pltpu.sync_copy(src_hbm.at[pl.ds(base, _CH)], sis[slot])
        pltpu.sync_copy(dst_hbm.at[pl.ds(base, _CH)], dis[slot])
        pltpu.async_copy(x_hbm.at[sis[slot]], srs[slot], sems[slot])
        pltpu.async_copy(x_hbm.at[dis[slot]], drs[slot], sems[slot])

    def _finish(c, slot):
        pltpu.make_async_copy(x_hbm.at[sis[slot]], srs[slot], sems[slot]).wait()
        pltpu.make_async_copy(x_hbm.at[dis[slot]], drs[slot], sems[slot]).wait()
        base = pl.multiple_of(base_w + c * _CH, 8)
        pltpu.sync_copy(srs[slot], outs_hbm.at[pl.ds(base, _CH)])
        pltpu.sync_copy(drs[slot], outd_hbm.at[pl.ds(base, _CH)])

    for slot in (0, 1):
        _start(slot, slot)

    def body(t, carry):
        for slot in (0, 1):
            c = 2 * t + slot
            _finish(c, slot)
            _start(c + 2, slot)
        return carry

    lax.fori_loop(0, (_EFULL - 2) // 2, body, 0)
    for slot in (0, 1):
        _finish(_EFULL - 2 + slot, slot)

    base = pl.multiple_of(base_w + _EFULL * _CH, 8)
    pltpu.sync_copy(src_hbm.at[pl.ds(base, _ETAIL)], si_t)
    pltpu.sync_copy(dst_hbm.at[pl.ds(base, _ETAIL)], di_t)
    cs = pltpu.async_copy(x_hbm.at[si_t], sr_t, sem0)
    cd = pltpu.async_copy(x_hbm.at[di_t], dr_t, sem0)
    cs.wait()
    cd.wait()
    pltpu.sync_copy(sr_t, outs_hbm.at[pl.ds(base, _ETAIL)])
    pltpu.sync_copy(dr_t, outd_hbm.at[pl.ds(base, _ETAIL)])


def _make_sc_scatter(with_values):
    scratch = [pltpu.VMEM((_CH,), jnp.int32),
               pltpu.VMEM((_CH,), jnp.int32),
               pltpu.VMEM((_ETAIL,), jnp.int32),
               pltpu.VMEM((_CH, _C), jnp.float32),
               pltpu.VMEM((_CH, _C), jnp.float32),
               pltpu.VMEM((_ETAIL, _C), jnp.float32),
               pltpu.VMEM_SHARED((_N, _C), jnp.float32),
               pltpu.SemaphoreType.DMA,
               pltpu.SemaphoreType.DMA]

    @functools.partial(
        pl.kernel, mesh=_mesh,
        out_type=[jax.ShapeDtypeStruct((_NC * _N, _C), jnp.float32)],
        scratch_types=scratch)
    def k(h_hbm, dst_hbm, z128_hbm, o128_hbm, agg_hbm,
          di0, di1, di_t, hv0, hv1, hv_t, acc_sh, sem0, sem1):
        cid = lax.axis_index("c")
        sid = lax.axis_index("s")
        wid = sid * _NC + cid
        dis = (di0, di1)
        hvs = (hv0, hv1)
        sems = (sem0, sem1)

        pltpu.sync_copy(z128_hbm, hv0)
        _RPT = 624
        _NT_FULL = _RPT // _CH
        _NT_TAIL = _RPT - _NT_FULL * _CH
        _GTB = _NS * _RPT
        nbase = sid * _RPT

        def zbody(t, carry):
            b = nbase + t * _CH
            pltpu.sync_copy(hv0, acc_sh.at[pl.ds(b, _CH)])
            return carry

        lax.fori_loop(0, _NT_FULL, zbody, 0)
        tb = nbase + _NT_FULL * _CH
        pltpu.sync_copy(hv0.at[pl.ds(0, _NT_TAIL)], acc_sh.at[pl.ds(tb, _NT_TAIL)])
        pltpu.sync_copy(hv0.at[pl.ds(0, _NTAIL)], acc_sh.at[pl.ds(_GTB, _NTAIL)])
        if not with_values:
            pltpu.sync_copy(o128_hbm, hv0)
            pltpu.sync_copy(o128_hbm, hv1)
            pltpu.sync_copy(o128_hbm.at[pl.ds(0, _ETAIL)], hv_t)

        plsc.subcore_barrier()

        base_w = pl.multiple_of(wid * _EPW, 8)

        def _stage(c, slot):
            base = pl.multiple_of(base_w + c * _CH, 8)
            pltpu.sync_copy(dst_hbm.at[pl.ds(base, _CH)], dis[slot])
            if with_values:
                pltpu.async_copy(h_hbm.at[pl.ds(base, _CH)], hvs[slot],
                                 sems[slot])

        def _commit(slot):
            if with_values:
                pltpu.make_async_copy(
                    h_hbm.at[pl.ds(0, _CH)], hvs[slot], sems[slot]).wait()
            pltpu.sync_copy(hvs[slot], acc_sh.at[dis[slot]], add=True)

        for slot in (0, 1):
            _stage(slot, slot)

        def body(t, carry):
            for slot in (0, 1):
                _commit(slot)
                _stage(2 * t + slot + 2, slot)
            return carry

        lax.fori_loop(0, (_EFULL - 2) // 2, body, 0)
        for slot in (0, 1):
            _commit(slot)

        base = pl.multiple_of(base_w + _EFULL * _CH, 8)
        pltpu.sync_copy(dst_hbm.at[pl.ds(base, _ETAIL)], di_t)
        if with_values:
            pltpu.sync_copy(h_hbm.at[pl.ds(base, _ETAIL)], hv_t)
        pltpu.sync_copy(hv_t, acc_sh.at[di_t], add=True)

        plsc.subcore_barrier()

        def obody(t, carry):
            b = nbase + t * _CH
            ob = cid * _N + b
            pltpu.sync_copy(acc_sh.at[pl.ds(b, _CH)], hv0)
            pltpu.sync_copy(hv0, agg_hbm.at[pl.ds(ob, _CH)])
            return carry

        lax.fori_loop(0, _NT_FULL, obody, 0)
        ob = cid * _N + tb
        pltpu.sync_copy(acc_sh.at[pl.ds(tb, _NT_TAIL)], hv0.at[pl.ds(0, _NT_TAIL)])
        pltpu.sync_copy(hv0.at[pl.ds(0, _NT_TAIL)], agg_hbm.at[pl.ds(ob, _NT_TAIL)])
        gob = cid * _N + _GTB
        pltpu.sync_copy(acc_sh.at[pl.ds(_GTB, _NTAIL)], hv_t)
        pltpu.sync_copy(hv_t, agg_hbm.at[pl.ds(gob, _NTAIL)])

    return k


def _unwrap(res):
    return res[0] if isinstance(res, (list, tuple)) else res


_sc_scatter_vals_raw = _make_sc_scatter(True)
_sc_deg_raw = _make_sc_scatter(False)


def _sc_scatter_vals(*a):
    return _unwrap(_sc_scatter_vals_raw(*a))


def _sc_deg(*a):
    return _unwrap(_sc_deg_raw(*a))


def _full_spec(arr):
    nd = arr.ndim
    return pl.BlockSpec(arr.shape, lambda i, _n=nd: (0,) * _n)


def _edge_spec(width):
    return pl.BlockSpec((_BE, width), lambda i: (i, 0))


def _bdot(a, w):
    return jnp.dot(a, w, preferred_element_type=jnp.float32)


def _mlp_tail(h, wrefs):
    n = len(wrefs) // 2
    for i in range(n):
        w = wrefs[2 * i][...]
        b = wrefs[2 * i + 1][...]
        h = _bdot(h, w) + b
        if i < n - 1:
            h = jnp.maximum(h, 0.0)
    return h


def _split_first(params, widths):
    w0 = params[0]
    parts = []
    off = 0
    for w in widths:
        parts.append(w0[off:off + w])
        off += w
    rest = []
    for i in range(1, len(params)):
        p = params[i]
        rest.append(p.reshape(1, -1) if p.ndim == 1 else p)
    return parts, rest


def _nc_mlp(xs, xd, ang, params):
    (ws, wd, wa), rest = _split_first(params, (_C, _C, 1))
    b0, tail = rest[0], rest[1:]
    ins = [xs, xd, ang, ws, wd, wa, b0] + tail
    n_tail = len(tail)

    def body(*refs):
        xs_r, xd_r, an_r, ws_r, wd_r, wa_r, b0_r = refs[:7]
        wrefs = refs[7:7 + n_tail]
        out_r = refs[7 + n_tail]
        h = (_bdot(xs_r[...], ws_r[...]) + _bdot(xd_r[...], wd_r[...])
             + an_r[...] * wa_r[...] + b0_r[...])
        h = jnp.maximum(h, 0.0)
        out_r[...] = _mlp_tail(h, wrefs)

    return pl.pallas_call(
        body,
        grid=(_E // _BE,),
        in_specs=[_edge_spec(_C), _edge_spec(_C), _edge_spec(1)]
                 + [_full_spec(a) for a in ins[3:]],
        out_specs=_edge_spec(_C),
        out_shape=jax.ShapeDtypeStruct((_E, _C), jnp.float32),
        compiler_params=pltpu.CompilerParams(
            dimension_semantics=("arbitrary",)),
    )(*ins)


def _fused_ec1_nc2(xs, xd, ang, act, ec1_params, nc2_params):
    (ews, ewd, ewa), erest = _split_first(ec1_params, (_C, _C, 1))
    eb0, etail = erest[0], erest[1:]
    (nws, nwd, nwa), nrest = _split_first(nc2_params, (_C, _C, 1))
    nb0, ntail = nrest[0], nrest[1:]
    ins = ([xs, xd, ang, act, ews, ewd, ewa, eb0] + etail
           + [nws, nwd, nwa, nb0] + ntail)
    ne, nn = len(etail), len(ntail)

    def body(*refs):
        xs_r, xd_r, an_r, ac_r = refs[:4]
        ews_r, ewd_r, ewa_r, eb0_r = refs[4:8]
        ewrefs = refs[8:8 + ne]
        nws_r, nwd_r, nwa_r, nb0_r = refs[8 + ne:12 + ne]
        nwrefs = refs[12 + ne:12 + ne + nn]
        e1_r, h2_r, ss_r = refs[12 + ne + nn:]
        xs_v = xs_r[...]
        xd_v = xd_r[...]
        he = (_bdot(xs_v, ews_r[...]) + _bdot(xd_v, ewd_r[...])
              + ac_r[...] * ewa_r[...] + eb0_r[...])
        he = jnp.maximum(he, 0.0)
        e1 = _mlp_tail(he, ewrefs)
        e1_r[...] = e1
        hn = (_bdot(xs_v, nws_r[...]) + _bdot(xd_v, nwd_r[...])
              + an_r[...] * nwa_r[...] + nb0_r[...])
        hn = jnp.maximum(hn, 0.0)
        h2_r[...] = _mlp_tail(hn, nwrefs)

        @pl.when(pl.program_id(0) == 0)
        def _():
            ss_r[...] = jnp.zeros((8, 128), jnp.float32)

        ss_r[...] += jnp.sum(e1 * e1)

    h_ec = ec1_params[-1].shape[0]
    return pl.pallas_call(
        body,
        grid=(_E // _BE,),
        in_specs=[_edge_spec(_C), _edge_spec(_C), _edge_spec(1), _edge_spec(1)]
                 + [_full_spec(a) for a in ins[4:]],
        out_specs=[_edge_spec(h_ec), _edge_spec(_C),
                   pl.BlockSpec((8, 128), lambda i: (0, 0))],
        out_shape=[jax.ShapeDtypeStruct((_E, h_ec), jnp.float32),
                   jax.ShapeDtypeStruct((_E, _C), jnp.float32),
                   jax.ShapeDtypeStruct((8, 128), jnp.float32)],
        compiler_params=pltpu.CompilerParams(
            dimension_semantics=("arbitrary",)),
    )(*ins)


def _ec2_mlp(xs, xd, e1, params):
    h_in = e1.shape[1]
    (ws, wd, we), rest = _split_first(params, (_C, _C, h_in))
    b0, tail = rest[0], rest[1:]
    ins = [xs, xd, e1, ws, wd, we, b0] + tail
    n_tail = len(tail)

    def body(*refs):
        xs_r, xd_r, e1_r, ws_r, wd_r, we_r, b0_r = refs[:7]
        wrefs = refs[7:7 + n_tail]
        out_r, ss_r = refs[7 + n_tail:]
        h = (_bdot(xs_r[...], ws_r[...]) + _bdot(xd_r[...], wd_r[...])
             + _bdot(e1_r[...], we_r[...]) + b0_r[...])
        h = jnp.maximum(h, 0.0)
        e2 = _mlp_tail(h, wrefs)
        out_r[...] = e2

        @pl.when(pl.program_id(0) == 0)
        def _():
            ss_r[...] = jnp.zeros((8, 128), jnp.float32)

        ss_r[...] += jnp.sum(e2 * e2)

    return pl.pallas_call(
        body,
        grid=(_E // _BE,),
        in_specs=[_edge_spec(_C), _edge_spec(_C), _edge_spec(h_in)]
                 + [_full_spec(a) for a in ins[3:]],
        out_specs=[_edge_spec(_C), pl.BlockSpec((8, 128), lambda i: (0, 0))],
        out_shape=[jax.ShapeDtypeStruct((_E, _C), jnp.float32),
                   jax.ShapeDtypeStruct((8, 128), jnp.float32)],
        compiler_params=pltpu.CompilerParams(
            dimension_semantics=("arbitrary",)),
    )(*ins)


_BN = 1000


def _node_update(x, agg_parts, deg_parts):
    nb = _N // _BN

    def body(x_r, a0_r, a1_r, d0_r, d1_r, out_r):
        deg = d0_r[...][:, 0:1] + d1_r[...][:, 0:1]
        deg = jnp.maximum(deg, 1.0)
        agg = a0_r[...] + a1_r[...]
        out_r[...] = x_r[...] + jnp.maximum(agg / deg, 0.0)

    return pl.pallas_call(
        body,
        grid=(nb,),
        in_specs=[pl.BlockSpec((_BN, _C), lambda i: (i, 0)),
                  pl.BlockSpec((_BN, _C), lambda i: (i, 0)),
                  pl.BlockSpec((_BN, _C), lambda i: (i + nb, 0)),
                  pl.BlockSpec((_BN, _C), lambda i: (i, 0)),
                  pl.BlockSpec((_BN, _C), lambda i: (i + nb, 0))],
        out_specs=pl.BlockSpec((_BN, _C), lambda i: (i, 0)),
        out_shape=jax.ShapeDtypeStruct((_N, _C), jnp.float32),
        compiler_params=pltpu.CompilerParams(
            dimension_semantics=("arbitrary",)),
    )(x, agg_parts, agg_parts, deg_parts, deg_parts)


def kernel(node_features, edge_index, angles, gt_edges, actions,
           nc1_params, ec1_params, nc2_params, ec2_params):
    src = edge_index[0]
    dst = edge_index[1]
    ang = angles.reshape(_E, 1)
    z128 = jnp.zeros((_CH, _C), jnp.float32)
    o128 = jnp.ones((_CH, _C), jnp.float32)

    x0 = node_features
    deg = _sc_deg(z128, dst, z128, o128)

    xs0, xd0 = _sc_gather2(x0, src, dst)
    h1 = _nc_mlp(xs0, xd0, ang, nc1_params)
    agg1 = _sc_scatter_vals(h1, dst, z128, o128)
    x1 = _node_update(x0, agg1, deg)

    xs1, xd1 = _sc_gather2(x1, src, dst)
    e1, h2, ss1 = _fused_ec1_nc2(xs1, xd1, ang, actions, ec1_params,
                                 nc2_params)
    agg2 = _sc_scatter_vals(h2, dst, z128, o128)
    x2 = _node_update(x1, agg2, deg)

    xs2, xd2 = _sc_gather2(x2, src, dst)
    e2, ss2 = _ec2_mlp(xs2, xd2, e1, ec2_params)

    h_ec = e1.shape[1]
    side = (ss1[0, 0] / (_E * float(h_ec)) + ss2[0, 0] / (_E * float(_C))) * 0.5
    return e2, side


# speedup vs baseline: 1.1113x; 1.0038x over previous
import functools

import jax
import jax.numpy as jnp
from jax import lax
from jax.experimental import pallas as pl
from jax.experimental.pallas import tpu as pltpu
from jax.experimental.pallas import tpu_sc as plsc

_N = 10000
_E = 320000
_C = 128

_NC = 2
_NS = 16
_NW = _NC * _NS
_CH = 128
_EPW = _E // _NW
_EFULL = _EPW // _CH
_ETAIL = _EPW - _EFULL * _CH
_NFULL = _N // _CH
_NTAIL = _N - _NFULL * _CH

_BE = 2000

_mesh = plsc.VectorSubcoreMesh(core_axis_name="c", subcore_axis_name="s")


@functools.partial(
    pl.kernel, mesh=_mesh,
    out_type=[jax.ShapeDtypeStruct((_E, _C), jnp.float32),
              jax.ShapeDtypeStruct((_E, _C), jnp.float32)],
    scratch_types=[pltpu.VMEM((_CH,), jnp.int32),
                   pltpu.VMEM((_CH,), jnp.int32),
                   pltpu.VMEM((_CH,), jnp.int32),
                   pltpu.VMEM((_CH,), jnp.int32),
                   pltpu.VMEM((_CH,), jnp.int32),
                   pltpu.VMEM((_CH,), jnp.int32),
                   pltpu.VMEM((_ETAIL,), jnp.int32),
                   pltpu.VMEM((_ETAIL,), jnp.int32),
                   pltpu.VMEM((_CH, _C), jnp.float32),
                   pltpu.VMEM((_CH, _C), jnp.float32),
                   pltpu.VMEM((_CH, _C), jnp.float32),
                   pltpu.VMEM((_CH, _C), jnp.float32),
                   pltpu.VMEM((_CH, _C), jnp.float32),
                   pltpu.VMEM((_CH, _C), jnp.float32),
                   pltpu.VMEM((_ETAIL, _C), jnp.float32),
                   pltpu.VMEM((_ETAIL, _C), jnp.float32),
                   pltpu.SemaphoreType.DMA,
                   pltpu.SemaphoreType.DMA,
                   pltpu.SemaphoreType.DMA,
                   pltpu.SemaphoreType.DMA,
                   pltpu.SemaphoreType.DMA,
                   pltpu.SemaphoreType.DMA],
)
def _sc_gather2(x_hbm, src_hbm, dst_hbm, outs_hbm, outd_hbm,
                si0, di0, si1, di1, si2, di2, si_t, di_t,
                sr0, dr0, sr1, dr1, sr2, dr2, sr_t, dr_t,
                g0, g1, g2, w0, w1, w2):
    wid = lax.axis_index("s") * _NC + lax.axis_index("c")
    base_w = pl.multiple_of(wid * _EPW, 8)
    sis = (si0, si1, si2)
    dis = (di0, di1, di2)
    srs = (sr0, sr1, sr2)
    drs = (dr0, dr1, dr2)
    gsems = (g0, g1, g2)
    wsems = (w0, w1, w2)

    def _start(c, slot, first):
        if not first:
            pltpu.make_async_copy(srs[slot], outs_hbm.at[pl.ds(0, _CH)],
                                  wsems[slot]).wait()
            pltpu.make_async_copy(drs[slot], outd_hbm.at[pl.ds(0, _CH)],
                                  wsems[slot]).wait()
        base = pl.multiple_of(base_w + c * _CH, 8)
        pltpu.sync_copy(src_hbm.at[pl.ds(base, _CH)], sis[slot])
        pltpu.sync_copy(dst_hbm.at[pl.ds(base, _CH)], dis[slot])
        pltpu.async_copy(x_hbm.at[sis[slot]], srs[slot], gsems[slot])
        pltpu.async_copy(x_hbm.at[dis[slot]], drs[slot], gsems[slot])

    def _finish(c, slot):
        pltpu.make_async_copy(x_hbm.at[sis[slot]], srs[slot],
                              gsems[slot]).wait()
        pltpu.make_async_copy(x_hbm.at[dis[slot]], drs[slot],
                              gsems[slot]).wait()
        base = pl.multiple_of(base_w + c * _CH, 8)
        pltpu.async_copy(srs[slot], outs_hbm.at[pl.ds(base, _CH)],
                         wsems[slot])
        pltpu.async_copy(drs[slot], outd_hbm.at[pl.ds(base, _CH)],
                         wsems[slot])

    for slot in (0, 1, 2):
        _start(slot, slot, True)

    def body(t, carry):
        for slot in (0, 1, 2):
            c = 3 * t + slot
            _finish(c, slot)
            _start(c + 3, slot, False)
        return carry

    lax.fori_loop(0, (_EFULL - 3) // 3, body, 0)
    for slot in (0, 1, 2):
        _finish(_EFULL - 3 + slot, slot)
    for slot in (0, 1, 2):
        pltpu.make_async_copy(srs[slot], outs_hbm.at[pl.ds(0, _CH)],
                              wsems[slot]).wait()
        pltpu.make_async_copy(drs[slot], outd_hbm.at[pl.ds(0, _CH)],
                              wsems[slot]).wait()

    base = pl.multiple_of(base_w + _EFULL * _CH, 8)
    pltpu.sync_copy(src_hbm.at[pl.ds(base, _ETAIL)], si_t)
    pltpu.sync_copy(dst_hbm.at[pl.ds(base, _ETAIL)], di_t)
    cs = pltpu.async_copy(x_hbm.at[si_t], sr_t, g0)
    cd = pltpu.async_copy(x_hbm.at[di_t], dr_t, g0)
    cs.wait()
    cd.wait()
    pltpu.sync_copy(sr_t, outs_hbm.at[pl.ds(base, _ETAIL)])
    pltpu.sync_copy(dr_t, outd_hbm.at[pl.ds(base, _ETAIL)])


def _make_sc_scatter(with_values):
    scratch = [pltpu.VMEM((_CH,), jnp.int32),
               pltpu.VMEM((_CH,), jnp.int32),
               pltpu.VMEM((_ETAIL,), jnp.int32),
               pltpu.VMEM((_CH, _C), jnp.float32),
               pltpu.VMEM((_CH, _C), jnp.float32),
               pltpu.VMEM((_ETAIL, _C), jnp.float32),
               pltpu.VMEM_SHARED((_N, _C), jnp.float32),
               pltpu.SemaphoreType.DMA,
               pltpu.SemaphoreType.DMA]

    @functools.partial(
        pl.kernel, mesh=_mesh,
        out_type=[jax.ShapeDtypeStruct((_NC * _N, _C), jnp.float32)],
        scratch_types=scratch)
    def k(h_hbm, dst_hbm, z128_hbm, o128_hbm, agg_hbm,
          di0, di1, di_t, hv0, hv1, hv_t, acc_sh, sem0, sem1):
        cid = lax.axis_index("c")
        sid = lax.axis_index("s")
        wid = sid * _NC + cid
        dis = (di0, di1)
        hvs = (hv0, hv1)
        sems = (sem0, sem1)

        pltpu.sync_copy(z128_hbm, hv0)
        _RPT = 624
        _NT_FULL = _RPT // _CH
        _NT_TAIL = _RPT - _NT_FULL * _CH
        _GTB = _NS * _RPT
        nbase = sid * _RPT

        def zbody(t, carry):
            b = nbase + t * _CH
            pltpu.sync_copy(hv0, acc_sh.at[pl.ds(b, _CH)])
            return carry

        lax.fori_loop(0, _NT_FULL, zbody, 0)
        tb = nbase + _NT_FULL * _CH
        pltpu.sync_copy(hv0.at[pl.ds(0, _NT_TAIL)], acc_sh.at[pl.ds(tb, _NT_TAIL)])
        pltpu.sync_copy(hv0.at[pl.ds(0, _NTAIL)], acc_sh.at[pl.ds(_GTB, _NTAIL)])
        if not with_values:
            pltpu.sync_copy(o128_hbm, hv0)
            pltpu.sync_copy(o128_hbm, hv1)
            pltpu.sync_copy(o128_hbm.at[pl.ds(0, _ETAIL)], hv_t)

        plsc.subcore_barrier()

        base_w = pl.multiple_of(wid * _EPW, 8)

        def _stage(c, slot):
            base = pl.multiple_of(base_w + c * _CH, 8)
            pltpu.sync_copy(dst_hbm.at[pl.ds(base, _CH)], dis[slot])
            if with_values:
                pltpu.async_copy(h_hbm.at[pl.ds(base, _CH)], hvs[slot],
                                 sems[slot])

        def _commit(slot):
            if with_values:
                pltpu.make_async_copy(
                    h_hbm.at[pl.ds(0, _CH)], hvs[slot], sems[slot]).wait()
            pltpu.sync_copy(hvs[slot], acc_sh.at[dis[slot]], add=True)

        for slot in (0, 1):
            _stage(slot, slot)

        def body(t, carry):
            for slot in (0, 1):
                _commit(slot)
                _stage(2 * t + slot + 2, slot)
            return carry

        lax.fori_loop(0, (_EFULL - 2) // 2, body, 0)
        for slot in (0, 1):
            _commit(slot)

        base = pl.multiple_of(base_w + _EFULL * _CH, 8)
        pltpu.sync_copy(dst_hbm.at[pl.ds(base, _ETAIL)], di_t)
        if with_values:
            pltpu.sync_copy(h_hbm.at[pl.ds(base, _ETAIL)], hv_t)
        pltpu.sync_copy(hv_t, acc_sh.at[di_t], add=True)

        plsc.subcore_barrier()

        def obody(t, carry):
            b = nbase + t * _CH
            ob = cid * _N + b
            pltpu.sync_copy(acc_sh.at[pl.ds(b, _CH)], hv0)
            pltpu.sync_copy(hv0, agg_hbm.at[pl.ds(ob, _CH)])
            return carry

        lax.fori_loop(0, _NT_FULL, obody, 0)
        ob = cid * _N + tb
        pltpu.sync_copy(acc_sh.at[pl.ds(tb, _NT_TAIL)], hv0.at[pl.ds(0, _NT_TAIL)])
        pltpu.sync_copy(hv0.at[pl.ds(0, _NT_TAIL)], agg_hbm.at[pl.ds(ob, _NT_TAIL)])
        gob = cid * _N + _GTB
        pltpu.sync_copy(acc_sh.at[pl.ds(_GTB, _NTAIL)], hv_t)
        pltpu.sync_copy(hv_t, agg_hbm.at[pl.ds(gob, _NTAIL)])

    return k


def _unwrap(res):
    return res[0] if isinstance(res, (list, tuple)) else res


_sc_scatter_vals_raw = _make_sc_scatter(True)
_sc_deg_raw = _make_sc_scatter(False)


def _sc_scatter_vals(*a):
    return _unwrap(_sc_scatter_vals_raw(*a))


def _sc_deg(*a):
    return _unwrap(_sc_deg_raw(*a))


def _full_spec(arr):
    nd = arr.ndim
    return pl.BlockSpec(arr.shape, lambda i, _n=nd: (0,) * _n)


def _edge_spec(width):
    return pl.BlockSpec((_BE, width), lambda i: (i, 0))


def _bdot(a, w):
    return jnp.dot(a, w, preferred_element_type=jnp.float32)


def _mlp_tail(h, wrefs):
    n = len(wrefs) // 2
    for i in range(n):
        w = wrefs[2 * i][...]
        b = wrefs[2 * i + 1][...]
        h = _bdot(h, w) + b
        if i < n - 1:
            h = jnp.maximum(h, 0.0)
    return h


def _split_first(params, widths):
    w0 = params[0]
    parts = []
    off = 0
    for w in widths:
        parts.append(w0[off:off + w])
        off += w
    rest = []
    for i in range(1, len(params)):
        p = params[i]
        rest.append(p.reshape(1, -1) if p.ndim == 1 else p)
    return parts, rest


def _nc_mlp(xs, xd, ang, params):
    (ws, wd, wa), rest = _split_first(params, (_C, _C, 1))
    b0, tail = rest[0], rest[1:]
    ins = [xs, xd, ang, ws, wd, wa, b0] + tail
    n_tail = len(tail)

    def body(*refs):
        xs_r, xd_r, an_r, ws_r, wd_r, wa_r, b0_r = refs[:7]
        wrefs = refs[7:7 + n_tail]
        out_r = refs[7 + n_tail]
        h = (_bdot(xs_r[...], ws_r[...]) + _bdot(xd_r[...], wd_r[...])
             + an_r[...] * wa_r[...] + b0_r[...])
        h = jnp.maximum(h, 0.0)
        out_r[...] = _mlp_tail(h, wrefs)

    return pl.pallas_call(
        body,
        grid=(_E // _BE,),
        in_specs=[_edge_spec(_C), _edge_spec(_C), _edge_spec(1)]
                 + [_full_spec(a) for a in ins[3:]],
        out_specs=_edge_spec(_C),
        out_shape=jax.ShapeDtypeStruct((_E, _C), jnp.float32),
        compiler_params=pltpu.CompilerParams(
            dimension_semantics=("arbitrary",)),
    )(*ins)


def _fused_ec1_nc2(xs, xd, ang, act, ec1_params, nc2_params):
    (ews, ewd, ewa), erest = _split_first(ec1_params, (_C, _C, 1))
    eb0, etail = erest[0], erest[1:]
    (nws, nwd, nwa), nrest = _split_first(nc2_params, (_C, _C, 1))
    nb0, ntail = nrest[0], nrest[1:]
    ins = ([xs, xd, ang, act, ews, ewd, ewa, eb0] + etail
           + [nws, nwd, nwa, nb0] + ntail)
    ne, nn = len(etail), len(ntail)

    def body(*refs):
        xs_r, xd_r, an_r, ac_r = refs[:4]
        ews_r, ewd_r, ewa_r, eb0_r = refs[4:8]
        ewrefs = refs[8:8 + ne]
        nws_r, nwd_r, nwa_r, nb0_r = refs[8 + ne:12 + ne]
        nwrefs = refs[12 + ne:12 + ne + nn]
        e1_r, h2_r, ss_r = refs[12 + ne + nn:]
        xs_v = xs_r[...]
        xd_v = xd_r[...]
        he = (_bdot(xs_v, ews_r[...]) + _bdot(xd_v, ewd_r[...])
              + ac_r[...] * ewa_r[...] + eb0_r[...])
        he = jnp.maximum(he, 0.0)
        e1 = _mlp_tail(he, ewrefs)
        e1_r[...] = e1
        hn = (_bdot(xs_v, nws_r[...]) + _bdot(xd_v, nwd_r[...])
              + an_r[...] * nwa_r[...] + nb0_r[...])
        hn = jnp.maximum(hn, 0.0)
        h2_r[...] = _mlp_tail(hn, nwrefs)

        @pl.when(pl.program_id(0) == 0)
        def _():
            ss_r[...] = jnp.zeros((8, 128), jnp.float32)

        ss_r[...] += jnp.sum(e1 * e1)

    h_ec = ec1_params[-1].shape[0]
    return pl.pallas_call(
        body,
        grid=(_E // _BE,),
        in_specs=[_edge_spec(_C), _edge_spec(_C), _edge_spec(1), _edge_spec(1)]
                 + [_full_spec(a) for a in ins[4:]],
        out_specs=[_edge_spec(h_ec), _edge_spec(_C),
                   pl.BlockSpec((8, 128), lambda i: (0, 0))],
        out_shape=[jax.ShapeDtypeStruct((_E, h_ec), jnp.float32),
                   jax.ShapeDtypeStruct((_E, _C), jnp.float32),
                   jax.ShapeDtypeStruct((8, 128), jnp.float32)],
        compiler_params=pltpu.CompilerParams(
            dimension_semantics=("arbitrary",)),
    )(*ins)


def _ec2_mlp(xs, xd, e1, params):
    h_in = e1.shape[1]
    (ws, wd, we), rest = _split_first(params, (_C, _C, h_in))
    b0, tail = rest[0], rest[1:]
    ins = [xs, xd, e1, ws, wd, we, b0] + tail
    n_tail = len(tail)

    def body(*refs):
        xs_r, xd_r, e1_r, ws_r, wd_r, we_r, b0_r = refs[:7]
        wrefs = refs[7:7 + n_tail]
        out_r, ss_r = refs[7 + n_tail:]
        h = (_bdot(xs_r[...], ws_r[...]) + _bdot(xd_r[...], wd_r[...])
             + _bdot(e1_r[...], we_r[...]) + b0_r[...])
        h = jnp.maximum(h, 0.0)
        e2 = _mlp_tail(h, wrefs)
        out_r[...] = e2

        @pl.when(pl.program_id(0) == 0)
        def _():
            ss_r[...] = jnp.zeros((8, 128), jnp.float32)

        ss_r[...] += jnp.sum(e2 * e2)

    return pl.pallas_call(
        body,
        grid=(_E // _BE,),
        in_specs=[_edge_spec(_C), _edge_spec(_C), _edge_spec(h_in)]
                 + [_full_spec(a) for a in ins[3:]],
        out_specs=[_edge_spec(_C), pl.BlockSpec((8, 128), lambda i: (0, 0))],
        out_shape=[jax.ShapeDtypeStruct((_E, _C), jnp.float32),
                   jax.ShapeDtypeStruct((8, 128), jnp.float32)],
        compiler_params=pltpu.CompilerParams(
            dimension_semantics=("arbitrary",)),
    )(*ins)


_BN = 1000


def _node_update(x, agg_parts, deg_parts):
    nb = _N // _BN

    def body(x_r, a0_r, a1_r, d0_r, d1_r, out_r):
        deg = d0_r[...][:, 0:1] + d1_r[...][:, 0:1]
        deg = jnp.maximum(deg, 1.0)
        agg = a0_r[...] + a1_r[...]
        out_r[...] = x_r[...] + jnp.maximum(agg / deg, 0.0)

    return pl.pallas_call(
        body,
        grid=(nb,),
        in_specs=[pl.BlockSpec((_BN, _C), lambda i: (i, 0)),
                  pl.BlockSpec((_BN, _C), lambda i: (i, 0)),
                  pl.BlockSpec((_BN, _C), lambda i: (i + nb, 0)),
                  pl.BlockSpec((_BN, _C), lambda i: (i, 0)),
                  pl.BlockSpec((_BN, _C), lambda i: (i + nb, 0))],
        out_specs=pl.BlockSpec((_BN, _C), lambda i: (i, 0)),
        out_shape=jax.ShapeDtypeStruct((_N, _C), jnp.float32),
        compiler_params=pltpu.CompilerParams(
            dimension_semantics=("arbitrary",)),
    )(x, agg_parts, agg_parts, deg_parts, deg_parts)


def kernel(node_features, edge_index, angles, gt_edges, actions,
           nc1_params, ec1_params, nc2_params, ec2_params):
    src = edge_index[0]
    dst = edge_index[1]
    ang = angles.reshape(_E, 1)
    z128 = jnp.zeros((_CH, _C), jnp.float32)
    o128 = jnp.ones((_CH, _C), jnp.float32)

    x0 = node_features
    deg = _sc_deg(z128, dst, z128, o128)

    xs0, xd0 = _sc_gather2(x0, src, dst)
    h1 = _nc_mlp(xs0, xd0, ang, nc1_params)
    agg1 = _sc_scatter_vals(h1, dst, z128, o128)
    x1 = _node_update(x0, agg1, deg)

    xs1, xd1 = _sc_gather2(x1, src, dst)
    e1, h2, ss1 = _fused_ec1_nc2(xs1, xd1, ang, actions, ec1_params,
                                 nc2_params)
    agg2 = _sc_scatter_vals(h2, dst, z128, o128)
    x2 = _node_update(x1, agg2, deg)

    xs2, xd2 = _sc_gather2(x2, src, dst)
    e2, ss2 = _ec2_mlp(xs2, xd2, e1, ec2_params)

    h_ec = e1.shape[1]
    side = (ss1[0, 0] / (_E * float(h_ec)) + ss2[0, 0] / (_E * float(_C))) * 0.5
    return e2, side


# speedup vs baseline: 1.1496x; 1.0345x over previous
import functools

import jax
import jax.numpy as jnp
from jax import lax
from jax.experimental import pallas as pl
from jax.experimental.pallas import tpu as pltpu
from jax.experimental.pallas import tpu_sc as plsc

_N = 10000
_E = 320000
_C = 128

_NC = 2
_NS = 16
_NW = _NC * _NS
_CH = 128
_EPW = _E // _NW
_EFULL = _EPW // _CH
_ETAIL = _EPW - _EFULL * _CH
_NFULL = _N // _CH
_NTAIL = _N - _NFULL * _CH

_BE = 2000

_mesh = plsc.VectorSubcoreMesh(core_axis_name="c", subcore_axis_name="s")


_GCH = 80
_GFULL = _EPW // _GCH


@functools.partial(
    pl.kernel, mesh=_mesh,
    out_type=[jax.ShapeDtypeStruct((_E, _C), jnp.float32),
              jax.ShapeDtypeStruct((_E, _C), jnp.float32)],
    scratch_types=[pltpu.VMEM((_GCH,), jnp.int32),
                   pltpu.VMEM((_GCH,), jnp.int32),
                   pltpu.VMEM((_GCH,), jnp.int32),
                   pltpu.VMEM((_GCH,), jnp.int32),
                   pltpu.VMEM((_GCH, _C), jnp.float32),
                   pltpu.VMEM((_GCH, _C), jnp.float32),
                   pltpu.VMEM((_GCH, _C), jnp.float32),
                   pltpu.VMEM((_GCH, _C), jnp.float32),
                   pltpu.VMEM_SHARED((_N, _C), jnp.float32),
                   pltpu.SemaphoreType.DMA,
                   pltpu.SemaphoreType.DMA,
                   pltpu.SemaphoreType.DMA,
                   pltpu.SemaphoreType.DMA],
)
def _sc_gather2(x_hbm, src_hbm, dst_hbm, outs_hbm, outd_hbm,
                si0, di0, si1, di1,
                sr0, dr0, sr1, dr1, xsh, g0, g1, w0, w1):
    wid = lax.axis_index("s") * _NC + lax.axis_index("c")
    sid = lax.axis_index("s")
    base_w = pl.multiple_of(wid * _EPW, 8)
    sis = (si0, si1)
    dis = (di0, di1)
    srs = (sr0, sr1)
    drs = (dr0, dr1)
    gsems = (g0, g1)
    wsems = (w0, w1)

    nbase = sid * 624

    def stage(t, carry):
        b = nbase + t * _GCH
        pltpu.sync_copy(x_hbm.at[pl.ds(b, _GCH)], sr0)
        pltpu.sync_copy(sr0, xsh.at[pl.ds(b, _GCH)])
        return carry

    lax.fori_loop(0, 7, stage, 0)
    tb = nbase + 560
    pltpu.sync_copy(x_hbm.at[pl.ds(tb, 64)], sr0.at[pl.ds(0, 64)])
    pltpu.sync_copy(sr0.at[pl.ds(0, 64)], xsh.at[pl.ds(tb, 64)])
    pltpu.sync_copy(x_hbm.at[pl.ds(9984, 16)], sr0.at[pl.ds(0, 16)])
    pltpu.sync_copy(sr0.at[pl.ds(0, 16)], xsh.at[pl.ds(9984, 16)])
    plsc.subcore_barrier()

    def _start(c, slot, first):
        if not first:
            pltpu.make_async_copy(srs[slot], outs_hbm.at[pl.ds(0, _GCH)],
                                  wsems[slot]).wait()
            pltpu.make_async_copy(drs[slot], outd_hbm.at[pl.ds(0, _GCH)],
                                  wsems[slot]).wait()
        base = pl.multiple_of(base_w + c * _GCH, 8)
        pltpu.sync_copy(src_hbm.at[pl.ds(base, _GCH)], sis[slot])
        pltpu.sync_copy(dst_hbm.at[pl.ds(base, _GCH)], dis[slot])
        pltpu.async_copy(xsh.at[sis[slot]], srs[slot], gsems[slot])
        pltpu.async_copy(xsh.at[dis[slot]], drs[slot], gsems[slot])

    def _finish(c, slot):
        pltpu.make_async_copy(xsh.at[sis[slot]], srs[slot],
                              gsems[slot]).wait()
        pltpu.make_async_copy(xsh.at[dis[slot]], drs[slot],
                              gsems[slot]).wait()
        base = pl.multiple_of(base_w + c * _GCH, 8)
        pltpu.async_copy(srs[slot], outs_hbm.at[pl.ds(base, _GCH)],
                         wsems[slot])
        pltpu.async_copy(drs[slot], outd_hbm.at[pl.ds(base, _GCH)],
                         wsems[slot])

    for slot in (0, 1):
        _start(slot, slot, True)

    def body(t, carry):
        for slot in (0, 1):
            c = 2 * t + slot
            _finish(c, slot)
            _start(c + 2, slot, False)
        return carry

    lax.fori_loop(0, (_GFULL - 3) // 2, body, 0)
    for slot in (0, 1):
        _finish(122 + slot, slot)
    _start(124, 0, False)
    _finish(124, 0)
    pltpu.make_async_copy(sr0, outs_hbm.at[pl.ds(0, _GCH)], w0).wait()
    pltpu.make_async_copy(dr0, outd_hbm.at[pl.ds(0, _GCH)], w0).wait()
    pltpu.make_async_copy(sr1, outs_hbm.at[pl.ds(0, _GCH)], w1).wait()
    pltpu.make_async_copy(dr1, outd_hbm.at[pl.ds(0, _GCH)], w1).wait()


def _make_sc_scatter(with_values):
    scratch = [pltpu.VMEM((_CH,), jnp.int32),
               pltpu.VMEM((_CH,), jnp.int32),
               pltpu.VMEM((_ETAIL,), jnp.int32),
               pltpu.VMEM((_CH, _C), jnp.float32),
               pltpu.VMEM((_CH, _C), jnp.float32),
               pltpu.VMEM((_ETAIL, _C), jnp.float32),
               pltpu.VMEM_SHARED((_N, _C), jnp.float32),
               pltpu.SemaphoreType.DMA,
               pltpu.SemaphoreType.DMA]

    @functools.partial(
        pl.kernel, mesh=_mesh,
        out_type=[jax.ShapeDtypeStruct((_NC * _N, _C), jnp.float32)],
        scratch_types=scratch)
    def k(h_hbm, dst_hbm, z128_hbm, o128_hbm, agg_hbm,
          di0, di1, di_t, hv0, hv1, hv_t, acc_sh, sem0, sem1):
        cid = lax.axis_index("c")
        sid = lax.axis_index("s")
        wid = sid * _NC + cid
        dis = (di0, di1)
        hvs = (hv0, hv1)
        sems = (sem0, sem1)

        pltpu.sync_copy(z128_hbm, hv0)
        _RPT = 624
        _NT_FULL = _RPT // _CH
        _NT_TAIL = _RPT - _NT_FULL * _CH
        _GTB = _NS * _RPT
        nbase = sid * _RPT

        def zbody(t, carry):
            b = nbase + t * _CH
            pltpu.sync_copy(hv0, acc_sh.at[pl.ds(b, _CH)])
            return carry

        lax.fori_loop(0, _NT_FULL, zbody, 0)
        tb = nbase + _NT_FULL * _CH
        pltpu.sync_copy(hv0.at[pl.ds(0, _NT_TAIL)], acc_sh.at[pl.ds(tb, _NT_TAIL)])
        pltpu.sync_copy(hv0.at[pl.ds(0, _NTAIL)], acc_sh.at[pl.ds(_GTB, _NTAIL)])
        if not with_values:
            pltpu.sync_copy(o128_hbm, hv0)
            pltpu.sync_copy(o128_hbm, hv1)
            pltpu.sync_copy(o128_hbm.at[pl.ds(0, _ETAIL)], hv_t)

        plsc.subcore_barrier()

        base_w = pl.multiple_of(wid * _EPW, 8)

        def _stage(c, slot):
            base = pl.multiple_of(base_w + c * _CH, 8)
            pltpu.sync_copy(dst_hbm.at[pl.ds(base, _CH)], dis[slot])
            if with_values:
                pltpu.async_copy(h_hbm.at[pl.ds(base, _CH)], hvs[slot],
                                 sems[slot])

        def _commit(slot):
            if with_values:
                pltpu.make_async_copy(
                    h_hbm.at[pl.ds(0, _CH)], hvs[slot], sems[slot]).wait()
            pltpu.sync_copy(hvs[slot], acc_sh.at[dis[slot]], add=True)

        for slot in (0, 1):
            _stage(slot, slot)

        def body(t, carry):
            for slot in (0, 1):
                _commit(slot)
                _stage(2 * t + slot + 2, slot)
            return carry

        lax.fori_loop(0, (_EFULL - 2) // 2, body, 0)
        for slot in (0, 1):
            _commit(slot)

        base = pl.multiple_of(base_w + _EFULL * _CH, 8)
        pltpu.sync_copy(dst_hbm.at[pl.ds(base, _ETAIL)], di_t)
        if with_values:
            pltpu.sync_copy(h_hbm.at[pl.ds(base, _ETAIL)], hv_t)
        pltpu.sync_copy(hv_t, acc_sh.at[di_t], add=True)

        plsc.subcore_barrier()

        def obody(t, carry):
            b = nbase + t * _CH
            ob = cid * _N + b
            pltpu.sync_copy(acc_sh.at[pl.ds(b, _CH)], hv0)
            pltpu.sync_copy(hv0, agg_hbm.at[pl.ds(ob, _CH)])
            return carry

        lax.fori_loop(0, _NT_FULL, obody, 0)
        ob = cid * _N + tb
        pltpu.sync_copy(acc_sh.at[pl.ds(tb, _NT_TAIL)], hv0.at[pl.ds(0, _NT_TAIL)])
        pltpu.sync_copy(hv0.at[pl.ds(0, _NT_TAIL)], agg_hbm.at[pl.ds(ob, _NT_TAIL)])
        gob = cid * _N + _GTB
        pltpu.sync_copy(acc_sh.at[pl.ds(_GTB, _NTAIL)], hv_t)
        pltpu.sync_copy(hv_t, agg_hbm.at[pl.ds(gob, _NTAIL)])

    return k


def _unwrap(res):
    return res[0] if isinstance(res, (list, tuple)) else res


_sc_scatter_vals_raw = _make_sc_scatter(True)
_sc_deg_raw = _make_sc_scatter(False)


def _sc_scatter_vals(*a):
    return _unwrap(_sc_scatter_vals_raw(*a))


def _sc_deg(*a):
    return _unwrap(_sc_deg_raw(*a))


def _full_spec(arr):
    nd = arr.ndim
    return pl.BlockSpec(arr.shape, lambda i, _n=nd: (0,) * _n)


def _edge_spec(width):
    return pl.BlockSpec((_BE, width), lambda i: (i, 0))


def _bdot(a, w):
    return jnp.dot(a, w, preferred_element_type=jnp.float32)


def _mlp_tail(h, wrefs):
    n = len(wrefs) // 2
    for i in range(n):
        w = wrefs[2 * i][...]
        b = wrefs[2 * i + 1][...]
        h = _bdot(h, w) + b
        if i < n - 1:
            h = jnp.maximum(h, 0.0)
    return h


def _split_first(params, widths):
    w0 = params[0]
    parts = []
    off = 0
    for w in widths:
        parts.append(w0[off:off + w])
        off += w
    rest = []
    for i in range(1, len(params)):
        p = params[i]
        rest.append(p.reshape(1, -1) if p.ndim == 1 else p)
    return parts, rest


def _nc_mlp(xs, xd, ang, params):
    (ws, wd, wa), rest = _split_first(params, (_C, _C, 1))
    b0, tail = rest[0], rest[1:]
    ins = [xs, xd, ang, ws, wd, wa, b0] + tail
    n_tail = len(tail)

    def body(*refs):
        xs_r, xd_r, an_r, ws_r, wd_r, wa_r, b0_r = refs[:7]
        wrefs = refs[7:7 + n_tail]
        out_r = refs[7 + n_tail]
        h = (_bdot(xs_r[...], ws_r[...]) + _bdot(xd_r[...], wd_r[...])
             + an_r[...] * wa_r[...] + b0_r[...])
        h = jnp.maximum(h, 0.0)
        out_r[...] = _mlp_tail(h, wrefs)

    return pl.pallas_call(
        body,
        grid=(_E // _BE,),
        in_specs=[_edge_spec(_C), _edge_spec(_C), _edge_spec(1)]
                 + [_full_spec(a) for a in ins[3:]],
        out_specs=_edge_spec(_C),
        out_shape=jax.ShapeDtypeStruct((_E, _C), jnp.float32),
        compiler_params=pltpu.CompilerParams(
            dimension_semantics=("arbitrary",)),
    )(*ins)


def _fused_ec1_nc2(xs, xd, ang, act, ec1_params, nc2_params):
    (ews, ewd, ewa), erest = _split_first(ec1_params, (_C, _C, 1))
    eb0, etail = erest[0], erest[1:]
    (nws, nwd, nwa), nrest = _split_first(nc2_params, (_C, _C, 1))
    nb0, ntail = nrest[0], nrest[1:]
    ins = ([xs, xd, ang, act, ews, ewd, ewa, eb0] + etail
           + [nws, nwd, nwa, nb0] + ntail)
    ne, nn = len(etail), len(ntail)

    def body(*refs):
        xs_r, xd_r, an_r, ac_r = refs[:4]
        ews_r, ewd_r, ewa_r, eb0_r = refs[4:8]
        ewrefs = refs[8:8 + ne]
        nws_r, nwd_r, nwa_r, nb0_r = refs[8 + ne:12 + ne]
        nwrefs = refs[12 + ne:12 + ne + nn]
        e1_r, h2_r, ss_r = refs[12 + ne + nn:]
        xs_v = xs_r[...]
        xd_v = xd_r[...]
        he = (_bdot(xs_v, ews_r[...]) + _bdot(xd_v, ewd_r[...])
              + ac_r[...] * ewa_r[...] + eb0_r[...])
        he = jnp.maximum(he, 0.0)
        e1 = _mlp_tail(he, ewrefs)
        e1_r[...] = e1
        hn = (_bdot(xs_v, nws_r[...]) + _bdot(xd_v, nwd_r[...])
              + an_r[...] * nwa_r[...] + nb0_r[...])
        hn = jnp.maximum(hn, 0.0)
        h2_r[...] = _mlp_tail(hn, nwrefs)

        @pl.when(pl.program_id(0) == 0)
        def _():
            ss_r[...] = jnp.zeros((8, 128), jnp.float32)

        ss_r[...] += jnp.sum(e1 * e1)

    h_ec = ec1_params[-1].shape[0]
    return pl.pallas_call(
        body,
        grid=(_E // _BE,),
        in_specs=[_edge_spec(_C), _edge_spec(_C), _edge_spec(1), _edge_spec(1)]
                 + [_full_spec(a) for a in ins[4:]],
        out_specs=[_edge_spec(h_ec), _edge_spec(_C),
                   pl.BlockSpec((8, 128), lambda i: (0, 0))],
        out_shape=[jax.ShapeDtypeStruct((_E, h_ec), jnp.float32),
                   jax.ShapeDtypeStruct((_E, _C), jnp.float32),
                   jax.ShapeDtypeStruct((8, 128), jnp.float32)],
        compiler_params=pltpu.CompilerParams(
            dimension_semantics=("arbitrary",)),
    )(*ins)


def _ec2_mlp(xs, xd, e1, params):
    h_in = e1.shape[1]
    (ws, wd, we), rest = _split_first(params, (_C, _C, h_in))
    b0, tail = rest[0], rest[1:]
    ins = [xs, xd, e1, ws, wd, we, b0] + tail
    n_tail = len(tail)

    def body(*refs):
        xs_r, xd_r, e1_r, ws_r, wd_r, we_r, b0_r = refs[:7]
        wrefs = refs[7:7 + n_tail]
        out_r, ss_r = refs[7 + n_tail:]
        h = (_bdot(xs_r[...], ws_r[...]) + _bdot(xd_r[...], wd_r[...])
             + _bdot(e1_r[...], we_r[...]) + b0_r[...])
        h = jnp.maximum(h, 0.0)
        e2 = _mlp_tail(h, wrefs)
        out_r[...] = e2

        @pl.when(pl.program_id(0) == 0)
        def _():
            ss_r[...] = jnp.zeros((8, 128), jnp.float32)

        ss_r[...] += jnp.sum(e2 * e2)

    return pl.pallas_call(
        body,
        grid=(_E // _BE,),
        in_specs=[_edge_spec(_C), _edge_spec(_C), _edge_spec(h_in)]
                 + [_full_spec(a) for a in ins[3:]],
        out_specs=[_edge_spec(_C), pl.BlockSpec((8, 128), lambda i: (0, 0))],
        out_shape=[jax.ShapeDtypeStruct((_E, _C), jnp.float32),
                   jax.ShapeDtypeStruct((8, 128), jnp.float32)],
        compiler_params=pltpu.CompilerParams(
            dimension_semantics=("arbitrary",)),
    )(*ins)


_BN = 1000


def _node_update(x, agg_parts, deg_parts):
    nb = _N // _BN

    def body(x_r, a0_r, a1_r, d0_r, d1_r, out_r):
        deg = d0_r[...][:, 0:1] + d1_r[...][:, 0:1]
        deg = jnp.maximum(deg, 1.0)
        agg = a0_r[...] + a1_r[...]
        out_r[...] = x_r[...] + jnp.maximum(agg / deg, 0.0)

    return pl.pallas_call(
        body,
        grid=(nb,),
        in_specs=[pl.BlockSpec((_BN, _C), lambda i: (i, 0)),
                  pl.BlockSpec((_BN, _C), lambda i: (i, 0)),
                  pl.BlockSpec((_BN, _C), lambda i: (i + nb, 0)),
                  pl.BlockSpec((_BN, _C), lambda i: (i, 0)),
                  pl.BlockSpec((_BN, _C), lambda i: (i + nb, 0))],
        out_specs=pl.BlockSpec((_BN, _C), lambda i: (i, 0)),
        out_shape=jax.ShapeDtypeStruct((_N, _C), jnp.float32),
        compiler_params=pltpu.CompilerParams(
            dimension_semantics=("arbitrary",)),
    )(x, agg_parts, agg_parts, deg_parts, deg_parts)


def kernel(node_features, edge_index, angles, gt_edges, actions,
           nc1_params, ec1_params, nc2_params, ec2_params):
    src = edge_index[0]
    dst = edge_index[1]
    ang = angles.reshape(_E, 1)
    z128 = jnp.zeros((_CH, _C), jnp.float32)
    o128 = jnp.ones((_CH, _C), jnp.float32)

    x0 = node_features
    deg = _sc_deg(z128, dst, z128, o128)

    xs0, xd0 = _sc_gather2(x0, src, dst)
    h1 = _nc_mlp(xs0, xd0, ang, nc1_params)
    agg1 = _sc_scatter_vals(h1, dst, z128, o128)
    x1 = _node_update(x0, agg1, deg)

    xs1, xd1 = _sc_gather2(x1, src, dst)
    e1, h2, ss1 = _fused_ec1_nc2(xs1, xd1, ang, actions, ec1_params,
                                 nc2_params)
    agg2 = _sc_scatter_vals(h2, dst, z128, o128)
    x2 = _node_update(x1, agg2, deg)

    xs2, xd2 = _sc_gather2(x2, src, dst)
    e2, ss2 = _ec2_mlp(xs2, xd2, e1, ec2_params)

    h_ec = e1.shape[1]
    side = (ss1[0, 0] / (_E * float(h_ec)) + ss2[0, 0] / (_E * float(_C))) * 0.5
    return e2, side
